# baseline (speedup 1.0000x reference)
import functools

import jax
import jax.numpy as jnp
from jax import lax
from jax.experimental import pallas as pl
from jax.experimental.pallas import tpu as pltpu
from jax.experimental.pallas import tpu_sc as plsc

N = 50000
E = 800000
HID = 64
NC, NS = 2, 16
NW = NC * NS
ROWS = E // 128
SB_FULL = ROWS // 8
TAIL_ROWS = ROWS - SB_FULL * 8
HALF = N // 2
CPAD = 25008
DUMMY = 25000
WB = 1562
GROUP = 16
NGRP = ROWS // GROUP
TAILR = ROWS - NGRP * GROUP

_MESH = dict(core_axis_name="c", subcore_axis_name="s", num_cores=NC,
             num_subcores=NS)


def _make_gather(specs, width, dtype=jnp.float32):
    n_out = len(specs)
    out_type = [jax.ShapeDtypeStruct((E, width), dtype)] * n_out

    @functools.partial(
        pl.kernel, out_type=out_type,
        mesh=plsc.VectorSubcoreMesh(**_MESH),
        compiler_params=pltpu.CompilerParams(use_tc_tiling_on_sc=False),
        scratch_types=[
            pltpu.VMEM((1024,), jnp.int32),
            pltpu.VMEM((1024, width), dtype),
            pltpu.SemaphoreType.DMA,
        ])
    def k(*refs):
        n_in = max(max(s[0], s[1]) for s in specs) + 1
        ins = refs[:n_in]
        outs = refs[n_in:n_in + n_out]
        idx_v, rows_v, sem = refs[n_in + n_out:]
        cid = lax.axis_index("c")
        sid = lax.axis_index("s")
        wid = sid * NC + cid

        def task(table, ei, row, out):
            def unit(nrows, sb):
                pltpu.sync_copy(ei.at[row, pl.ds(sb * 1024, nrows * 128)],
                                idx_v.at[pl.ds(0, nrows * 128)])
                descs = [pltpu.async_copy(
                    table.at[idx_v.at[pl.ds(j * 128, 128)]],
                    rows_v.at[pl.ds(j * 128, 128)], sem)
                         for j in range(nrows)]
                for d in descs:
                    d.wait()
                pltpu.sync_copy(rows_v.at[pl.ds(0, nrows * 128)],
                                out.at[pl.ds(sb * 1024, nrows * 128)])

            def body(it, carry):
                sb = it * NW + wid

                @pl.when(sb < SB_FULL)
                def _():
                    unit(8, sb)

                @pl.when(sb == SB_FULL)
                def _():
                    unit(TAIL_ROWS, sb)
                return carry

            lax.fori_loop(0, (SB_FULL + NW) // NW + 1, body, 0)

        for (ti, ei_i, row), out in zip(specs, outs):
            task(ins[ti], ins[ei_i], row, out)

    return k


def _scatter_kernel(with_count):
    out_type = [jax.ShapeDtypeStruct((N, HID), jnp.bfloat16)]
    scratch = [
        pltpu.VMEM((GROUP * 128,), jnp.int32),
        pltpu.VMEM((GROUP, 128), jnp.int32),
        pltpu.VMEM((128, HID), jnp.bfloat16),
        pltpu.VMEM_SHARED((CPAD, HID), jnp.bfloat16),
    ]
    if with_count:
        out_type.append(jax.ShapeDtypeStruct((N, 8), jnp.float32))
        scratch += [pltpu.VMEM((128, 8), jnp.float32),
                    pltpu.VMEM_SHARED((CPAD, 8), jnp.float32)]

    @functools.partial(
        pl.kernel,
        out_type=out_type,
        mesh=plsc.VectorSubcoreMesh(**_MESH),
        compiler_params=pltpu.CompilerParams(use_tc_tiling_on_sc=False),
        scratch_types=scratch)
    def k(msg, ei, zer, *rest):
        if with_count:
            (zer8, ones8, out, out_c, idx1_v, idx_v, msg_v, acc, ones_v,
             acc_c) = rest
        else:
            out, idx1_v, idx_v, msg_v, acc = rest
        cid = lax.axis_index("c")
        sid = lax.axis_index("s")
        base = cid * HALF

        z0 = sid * 1564
        zlast = CPAD - 1564 * (NS - 1)

        @pl.when(sid < NS - 1)
        def _():
            pltpu.sync_copy(zer.at[pl.ds(z0, 1564)], acc.at[pl.ds(z0, 1564)])
            if with_count:
                pltpu.sync_copy(zer8.at[pl.ds(z0, 1564)],
                                acc_c.at[pl.ds(z0, 1564)])

        @pl.when(sid == NS - 1)
        def _():
            pltpu.sync_copy(zer.at[pl.ds(z0, zlast)], acc.at[pl.ds(z0, zlast)])
            if with_count:
                pltpu.sync_copy(zer8.at[pl.ds(z0, zlast)],
                                acc_c.at[pl.ds(z0, zlast)])

        if with_count:
            pltpu.sync_copy(ones8, ones_v)
        plsc.subcore_barrier()

        def unit(nrows, g0):
            pltpu.sync_copy(ei.at[1, pl.ds(g0 * 128, nrows * 128)],
                            idx1_v.at[pl.ds(0, nrows * 128)])
            for r in range(nrows):
                for c in range(128 // 16):
                    v = idx1_v[pl.ds(r * 128 + c * 16, 16)]
                    l = v - base
                    ok = (l >= 0) & (l < HALF)
                    idx_v[r, pl.ds(c * 16, 16)] = jnp.where(ok, l, DUMMY)
            for r in range(nrows):
                pltpu.sync_copy(msg.at[pl.ds((g0 + r) * 128, 128)], msg_v)
                pltpu.sync_copy(msg_v, acc.at[idx_v.at[r]], add=True)
                if with_count:
                    pltpu.sync_copy(ones_v, acc_c.at[idx_v.at[r]], add=True)

        def body(it, carry):
            g = it * NS + sid

            @pl.when(g < NGRP)
            def _():
                unit(GROUP, g * GROUP)
            return carry

        lax.fori_loop(0, (NGRP + NS - 1) // NS, body, 0)

        @pl.when(sid < TAILR)
        def _():
            unit(1, NGRP * GROUP + sid)

        plsc.subcore_barrier()

        off = sid * WB
        pltpu.sync_copy(acc.at[pl.ds(off, WB)], out.at[pl.ds(base + off, WB)])
        if with_count:
            pltpu.sync_copy(acc_c.at[pl.ds(off, WB)],
                            out_c.at[pl.ds(base + off, WB)])

        @pl.when(sid == NS - 1)
        def _():
            pltpu.sync_copy(acc.at[pl.ds(WB * NS, HALF - WB * NS)],
                            out.at[pl.ds(base + WB * NS, HALF - WB * NS)])
            if with_count:
                pltpu.sync_copy(acc_c.at[pl.ds(WB * NS, HALF - WB * NS)],
                                out_c.at[pl.ds(base + WB * NS, HALF - WB * NS)])

    return k


def _node_enc_call(u, h, pu, ph):
    B = 2000

    def body(u_ref, h_ref, wu1, wu2, wu3, wh1, wh2, wh3, b1, b2, b3,
             eu_ref, eh_ref):
        x = jnp.tanh(jnp.dot(u_ref[...], wu1[...],
                             preferred_element_type=jnp.float32))
        x = jnp.tanh(jnp.dot(x, wu2[...], preferred_element_type=jnp.float32))
        eu_ref[...] = jnp.dot(
            x, wu3[...], preferred_element_type=jnp.float32
        ).astype(jnp.bfloat16)
        y = jnp.tanh(jnp.dot(h_ref[...], wh1[...],
                             preferred_element_type=jnp.float32) + b1[...])
        y = jnp.tanh(jnp.dot(y, wh2[...],
                             preferred_element_type=jnp.float32) + b2[...])
        eh_ref[...] = (jnp.dot(y, wh3[...], preferred_element_type=jnp.float32)
                       + b3[...]).astype(jnp.bfloat16)

    def full(shape):
        return pl.BlockSpec(shape, lambda i: (0,) * len(shape))

    return pl.pallas_call(
        body,
        grid=(N // B,),
        in_specs=[pl.BlockSpec((B, 16), lambda i: (i, 0)),
                  pl.BlockSpec((B, HID), lambda i: (i, 0)),
                  full((16, HID)), full((HID, HID)), full((HID, HID)),
                  full((HID, HID)), full((HID, HID)), full((HID, HID)),
                  full((1, HID)), full((1, HID)), full((1, HID))],
        out_specs=[pl.BlockSpec((B, HID), lambda i: (i, 0))] * 2,
        out_shape=[jax.ShapeDtypeStruct((N, HID), jnp.bfloat16)] * 2,
    )(u, h, pu[0]["W"], pu[1]["W"], pu[2]["W"],
      ph[0]["W"], ph[1]["W"], ph[2]["W"],
      ph[0]["b"].reshape(1, HID), ph[1]["b"].reshape(1, HID),
      ph[2]["b"].reshape(1, HID))


def _gate_call(psrc, pdst, dis, encg, pdis):
    B = 4000

    def body(pa, pd, ds_, eg, w1, b1, w2, b2, w3, b3, out):
        x = (pa[:, 0:1] * w1[0:1, :] + pa[:, 1:2] * w1[1:2, :]
             + pd[:, 0:1] * w1[2:3, :] + pd[:, 1:2] * w1[3:4, :]
             + ds_[...] * w1[4:5, :] + b1[...])
        x = jnp.tanh(x)
        x = jnp.tanh(jnp.dot(x, w2[...],
                             preferred_element_type=jnp.float32) + b2[...])
        g = jax.nn.sigmoid(jnp.dot(x, w3[...],
                                   preferred_element_type=jnp.float32)
                           + b3[...])
        out[...] = (g * eg[...].astype(jnp.float32)).astype(jnp.bfloat16)

    def full(shape):
        return pl.BlockSpec(shape, lambda i: (0,) * len(shape))

    return pl.pallas_call(
        body,
        grid=(E // B,),
        in_specs=[pl.BlockSpec((B, 8), lambda i: (i, 0)),
                  pl.BlockSpec((B, 8), lambda i: (i, 0)),
                  pl.BlockSpec((B, 1), lambda i: (i, 0)),
                  pl.BlockSpec((B, HID), lambda i: (i, 0)),
                  full((5, HID)), full((1, HID)),
                  full((HID, HID)), full((1, HID)),
                  full((HID, HID)), full((1, HID))],
        out_specs=pl.BlockSpec((B, HID), lambda i: (i, 0)),
        out_shape=jax.ShapeDtypeStruct((E, HID), jnp.bfloat16),
    )(psrc, pdst, dis, encg,
      pdis[0]["W"], pdis[0]["b"].reshape(1, HID),
      pdis[1]["W"], pdis[1]["b"].reshape(1, HID),
      pdis[2]["W"], pdis[2]["b"].reshape(1, HID))


def _final_call(pos_state, h, accA, accS, cntS, pup):
    B = 2000
    W1 = pup[0]["W"]
    wp, wh = W1[0:2], W1[2:2 + HID]
    wu, wm = W1[2 + HID:2 + 2 * HID], W1[2 + 2 * HID:2 + 3 * HID]

    def body(ps, h_ref, aA, aS, cS, wp_r, wh_r, wu_r, wm_r, b1, w2, b2, w3,
             b3, out):
        cnt = jnp.maximum(cS[:, 0:1], 1.0)
        aSf = aS[...].astype(jnp.float32)
        aAf = aA[...].astype(jnp.float32)
        mh = aSf / cnt
        x = (ps[:, 0:1] * wp_r[0:1, :] + ps[:, 1:2] * wp_r[1:2, :]
             + jnp.dot(h_ref[...], wh_r[...],
                       preferred_element_type=jnp.float32)
             + jnp.dot(aAf, wu_r[...], preferred_element_type=jnp.float32)
             + jnp.dot(mh, wm_r[...], preferred_element_type=jnp.float32)
             + b1[...])
        x = jnp.tanh(x)
        x = jnp.tanh(jnp.dot(x, w2[...],
                             preferred_element_type=jnp.float32) + b2[...])
        out[...] = jnp.dot(x, w3[...],
                           preferred_element_type=jnp.float32) + b3[...]

    def full(shape):
        return pl.BlockSpec(shape, lambda i: (0,) * len(shape))

    return pl.pallas_call(
        body,
        grid=(N // B,),
        in_specs=[pl.BlockSpec((B, 2), lambda i: (i, 0)),
                  pl.BlockSpec((B, HID), lambda i: (i, 0)),
                  pl.BlockSpec((B, HID), lambda i: (i, 0)),
                  pl.BlockSpec((B, HID), lambda i: (i, 0)),
                  pl.BlockSpec((B, 8), lambda i: (i, 0)),
                  full((2, HID)), full((HID, HID)), full((HID, HID)),
                  full((HID, HID)), full((1, HID)),
                  full((HID, HID)), full((1, HID)),
                  full((HID, HID)), full((1, HID))],
        out_specs=pl.BlockSpec((B, HID), lambda i: (i, 0)),
        out_shape=jax.ShapeDtypeStruct((N, HID), jnp.float32),
    )(pos_state, h, accA, accS, cntS,
      wp, wh, wu, wm, pup[0]["b"].reshape(1, HID),
      pup[1]["W"], pup[1]["b"].reshape(1, HID),
      pup[2]["W"], pup[2]["b"].reshape(1, HID))


_gather_pos = _make_gather(
    [(0, 2, 0), (1, 2, 1), (1, 3, 0), (1, 3, 1)], 8)
_gather_enc = _make_gather([(0, 1, 0)], HID, jnp.bfloat16)
_scatter_plain = _scatter_kernel(False)
_scatter_count = _scatter_kernel(True)


def kernel(h, u, pos_state, pos_action, dis_a2s, dis_s2s, a2s_edge_index,
           s2s_edge_index, params):
    pa_pad = jnp.pad(pos_action, ((0, 0), (0, 6)))
    ps_pad = jnp.pad(pos_state, ((0, 0), (0, 6)))
    zer64 = jnp.zeros((CPAD, HID), jnp.bfloat16)
    zer8 = jnp.zeros((CPAD, 8), jnp.float32)
    ones8 = jnp.zeros((128, 8), jnp.float32).at[:, 0].set(1.0)

    paA, pdA, psS, pdS = _gather_pos(pa_pad, ps_pad, a2s_edge_index,
                                     s2s_edge_index)
    enc_u, enc_h = _node_enc_call(u, h, params["u2h_enc_u"],
                                  params["h2h_enc_h"])
    guA, = _gather_enc(enc_u, a2s_edge_index)
    msgA = _gate_call(paA, pdA, dis_a2s, guA, params["u2h_enc_dis"])
    ghS, = _gather_enc(enc_h, s2s_edge_index)
    accA, = _scatter_plain(msgA, a2s_edge_index, zer64)
    msgS = _gate_call(psS, pdS, dis_s2s, ghS, params["h2h_enc_dis"])
    accS, cntS = _scatter_count(msgS, s2s_edge_index, zer64, zer8, ones8)
    return _final_call(pos_state, h, accA, accS, cntS, params["h_updater"])


# speedup vs baseline: 1.1499x; 1.1499x over previous
import functools

import jax
import jax.numpy as jnp
from jax import lax
from jax.experimental import pallas as pl
from jax.experimental.pallas import tpu as pltpu
from jax.experimental.pallas import tpu_sc as plsc

N = 50000
E = 800000
HID = 64
NC, NS = 2, 16
NW = NC * NS
ROWS = E // 128
SB_FULL = ROWS // 8
TAIL_ROWS = ROWS - SB_FULL * 8
HALF = N // 2
CPAD = 25008
DUMMY = 25000
WB = 1562
GROUP = 16
NGRP = ROWS // GROUP
TAILR = ROWS - NGRP * GROUP

_MESH = dict(core_axis_name="c", subcore_axis_name="s", num_cores=NC,
             num_subcores=NS)


def _make_gather(specs, width, dtype=jnp.float32):
    n_out = len(specs)
    out_type = [jax.ShapeDtypeStruct((E, width), dtype)] * n_out

    @functools.partial(
        pl.kernel, out_type=out_type,
        mesh=plsc.VectorSubcoreMesh(**_MESH),
        compiler_params=pltpu.CompilerParams(use_tc_tiling_on_sc=False),
        scratch_types=[
            pltpu.VMEM((1024,), jnp.int32),
            pltpu.VMEM((1024, width), dtype),
            pltpu.SemaphoreType.DMA,
        ])
    def k(*refs):
        n_in = max(max(s[0], s[1]) for s in specs) + 1
        ins = refs[:n_in]
        outs = refs[n_in:n_in + n_out]
        idx_v, rows_v, sem = refs[n_in + n_out:]
        cid = lax.axis_index("c")
        sid = lax.axis_index("s")
        wid = sid * NC + cid

        def task(table, ei, row, out):
            def unit(nrows, sb):
                pltpu.sync_copy(ei.at[row, pl.ds(sb * 1024, nrows * 128)],
                                idx_v.at[pl.ds(0, nrows * 128)])
                descs = [pltpu.async_copy(
                    table.at[idx_v.at[pl.ds(j * 128, 128)]],
                    rows_v.at[pl.ds(j * 128, 128)], sem)
                         for j in range(nrows)]
                for d in descs:
                    d.wait()
                pltpu.sync_copy(rows_v.at[pl.ds(0, nrows * 128)],
                                out.at[pl.ds(sb * 1024, nrows * 128)])

            def body(it, carry):
                sb = it * NW + wid

                @pl.when(sb < SB_FULL)
                def _():
                    unit(8, sb)

                @pl.when(sb == SB_FULL)
                def _():
                    unit(TAIL_ROWS, sb)
                return carry

            lax.fori_loop(0, (SB_FULL + NW) // NW + 1, body, 0)

        for (ti, ei_i, row), out in zip(specs, outs):
            task(ins[ti], ins[ei_i], row, out)

    return k


def _scatter_kernel(with_count):
    out_type = [jax.ShapeDtypeStruct((N, HID), jnp.float32)]
    scratch = [
        pltpu.VMEM((GROUP * 128,), jnp.int32),
        pltpu.VMEM((GROUP, 128), jnp.int32),
        pltpu.VMEM((128, HID), jnp.float32),
        pltpu.VMEM_SHARED((CPAD, HID), jnp.float32),
    ]
    if with_count:
        out_type.append(jax.ShapeDtypeStruct((N, 8), jnp.float32))
        scratch += [pltpu.VMEM((128, 8), jnp.float32),
                    pltpu.VMEM_SHARED((CPAD, 8), jnp.float32)]

    @functools.partial(
        pl.kernel,
        out_type=out_type,
        mesh=plsc.VectorSubcoreMesh(**_MESH),
        compiler_params=pltpu.CompilerParams(use_tc_tiling_on_sc=False),
        scratch_types=scratch)
    def k(msg, ei, zer, *rest):
        if with_count:
            (zer8, ones8, out, out_c, idx1_v, idx_v, msg_v, acc, ones_v,
             acc_c) = rest
        else:
            out, idx1_v, idx_v, msg_v, acc = rest
        cid = lax.axis_index("c")
        sid = lax.axis_index("s")
        base = cid * HALF

        z0 = sid * 1564
        zlast = CPAD - 1564 * (NS - 1)

        @pl.when(sid < NS - 1)
        def _():
            pltpu.sync_copy(zer.at[pl.ds(z0, 1564)], acc.at[pl.ds(z0, 1564)])
            if with_count:
                pltpu.sync_copy(zer8.at[pl.ds(z0, 1564)],
                                acc_c.at[pl.ds(z0, 1564)])

        @pl.when(sid == NS - 1)
        def _():
            pltpu.sync_copy(zer.at[pl.ds(z0, zlast)], acc.at[pl.ds(z0, zlast)])
            if with_count:
                pltpu.sync_copy(zer8.at[pl.ds(z0, zlast)],
                                acc_c.at[pl.ds(z0, zlast)])

        if with_count:
            pltpu.sync_copy(ones8, ones_v)
        plsc.subcore_barrier()

        def unit(nrows, g0):
            pltpu.sync_copy(ei.at[1, pl.ds(g0 * 128, nrows * 128)],
                            idx1_v.at[pl.ds(0, nrows * 128)])
            for r in range(nrows):
                for c in range(128 // 16):
                    v = idx1_v[pl.ds(r * 128 + c * 16, 16)]
                    l = v - base
                    ok = (l >= 0) & (l < HALF)
                    idx_v[r, pl.ds(c * 16, 16)] = jnp.where(ok, l, DUMMY)
            for r in range(nrows):
                pltpu.sync_copy(msg.at[pl.ds((g0 + r) * 128, 128)], msg_v)
                pltpu.sync_copy(msg_v, acc.at[idx_v.at[r]], add=True)
                if with_count:
                    pltpu.sync_copy(ones_v, acc_c.at[idx_v.at[r]], add=True)

        def body(it, carry):
            g = it * NS + sid

            @pl.when(g < NGRP)
            def _():
                unit(GROUP, g * GROUP)
            return carry

        lax.fori_loop(0, (NGRP + NS - 1) // NS, body, 0)

        @pl.when(sid < TAILR)
        def _():
            unit(1, NGRP * GROUP + sid)

        plsc.subcore_barrier()

        off = sid * WB
        pltpu.sync_copy(acc.at[pl.ds(off, WB)], out.at[pl.ds(base + off, WB)])
        if with_count:
            pltpu.sync_copy(acc_c.at[pl.ds(off, WB)],
                            out_c.at[pl.ds(base + off, WB)])

        @pl.when(sid == NS - 1)
        def _():
            pltpu.sync_copy(acc.at[pl.ds(WB * NS, HALF - WB * NS)],
                            out.at[pl.ds(base + WB * NS, HALF - WB * NS)])
            if with_count:
                pltpu.sync_copy(acc_c.at[pl.ds(WB * NS, HALF - WB * NS)],
                                out_c.at[pl.ds(base + WB * NS, HALF - WB * NS)])

    return k


def _node_enc_call(u, h, pu, ph):
    B = 2000

    def body(u_ref, h_ref, wu1, wu2, wu3, wh1, wh2, wh3, b1, b2, b3,
             eu_ref, eh_ref):
        x = jnp.tanh(jnp.dot(u_ref[...], wu1[...],
                             preferred_element_type=jnp.float32))
        x = jnp.tanh(jnp.dot(x, wu2[...], preferred_element_type=jnp.float32))
        eu_ref[...] = jnp.dot(x, wu3[...], preferred_element_type=jnp.float32)
        y = jnp.tanh(jnp.dot(h_ref[...], wh1[...],
                             preferred_element_type=jnp.float32) + b1[...])
        y = jnp.tanh(jnp.dot(y, wh2[...],
                             preferred_element_type=jnp.float32) + b2[...])
        eh_ref[...] = jnp.dot(y, wh3[...],
                              preferred_element_type=jnp.float32) + b3[...]

    def full(shape):
        return pl.BlockSpec(shape, lambda i: (0,) * len(shape))

    return pl.pallas_call(
        body,
        grid=(N // B,),
        in_specs=[pl.BlockSpec((B, 16), lambda i: (i, 0)),
                  pl.BlockSpec((B, HID), lambda i: (i, 0)),
                  full((16, HID)), full((HID, HID)), full((HID, HID)),
                  full((HID, HID)), full((HID, HID)), full((HID, HID)),
                  full((1, HID)), full((1, HID)), full((1, HID))],
        out_specs=[pl.BlockSpec((B, HID), lambda i: (i, 0))] * 2,
        out_shape=[jax.ShapeDtypeStruct((N, HID), jnp.float32)] * 2,
    )(u, h, pu[0]["W"], pu[1]["W"], pu[2]["W"],
      ph[0]["W"], ph[1]["W"], ph[2]["W"],
      ph[0]["b"].reshape(1, HID), ph[1]["b"].reshape(1, HID),
      ph[2]["b"].reshape(1, HID))


def _gate_call(psrc, pdst, dis, encg, pdis):
    B = 4000

    def body(pa, pd, ds_, eg, w1, b1, w2, b2, w3, b3, out):
        x = (pa[:, 0:1] * w1[0:1, :] + pa[:, 1:2] * w1[1:2, :]
             + pd[:, 0:1] * w1[2:3, :] + pd[:, 1:2] * w1[3:4, :]
             + ds_[...] * w1[4:5, :] + b1[...])
        x = jnp.tanh(x)
        x = jnp.tanh(jnp.dot(x, w2[...],
                             preferred_element_type=jnp.float32) + b2[...])
        g = jax.nn.sigmoid(jnp.dot(x, w3[...],
                                   preferred_element_type=jnp.float32)
                           + b3[...])
        out[...] = g * eg[...]

    def full(shape):
        return pl.BlockSpec(shape, lambda i: (0,) * len(shape))

    return pl.pallas_call(
        body,
        grid=(E // B,),
        in_specs=[pl.BlockSpec((B, 8), lambda i: (i, 0)),
                  pl.BlockSpec((B, 8), lambda i: (i, 0)),
                  pl.BlockSpec((B, 1), lambda i: (i, 0)),
                  pl.BlockSpec((B, HID), lambda i: (i, 0)),
                  full((5, HID)), full((1, HID)),
                  full((HID, HID)), full((1, HID)),
                  full((HID, HID)), full((1, HID))],
        out_specs=pl.BlockSpec((B, HID), lambda i: (i, 0)),
        out_shape=jax.ShapeDtypeStruct((E, HID), jnp.float32),
    )(psrc, pdst, dis, encg,
      pdis[0]["W"], pdis[0]["b"].reshape(1, HID),
      pdis[1]["W"], pdis[1]["b"].reshape(1, HID),
      pdis[2]["W"], pdis[2]["b"].reshape(1, HID))


def _final_call(pos_state, h, accA, accS, cntS, pup):
    B = 2000
    W1 = pup[0]["W"]
    wp, wh = W1[0:2], W1[2:2 + HID]
    wu, wm = W1[2 + HID:2 + 2 * HID], W1[2 + 2 * HID:2 + 3 * HID]

    def body(ps, h_ref, aA, aS, cS, wp_r, wh_r, wu_r, wm_r, b1, w2, b2, w3,
             b3, out):
        cnt = jnp.maximum(cS[:, 0:1], 1.0)
        aSf = aS[...]
        aAf = aA[...]
        mh = aSf / cnt
        x = (ps[:, 0:1] * wp_r[0:1, :] + ps[:, 1:2] * wp_r[1:2, :]
             + jnp.dot(h_ref[...], wh_r[...],
                       preferred_element_type=jnp.float32)
             + jnp.dot(aAf, wu_r[...], preferred_element_type=jnp.float32)
             + jnp.dot(mh, wm_r[...], preferred_element_type=jnp.float32)
             + b1[...])
        x = jnp.tanh(x)
        x = jnp.tanh(jnp.dot(x, w2[...],
                             preferred_element_type=jnp.float32) + b2[...])
        out[...] = jnp.dot(x, w3[...],
                           preferred_element_type=jnp.float32) + b3[...]

    def full(shape):
        return pl.BlockSpec(shape, lambda i: (0,) * len(shape))

    return pl.pallas_call(
        body,
        grid=(N // B,),
        in_specs=[pl.BlockSpec((B, 2), lambda i: (i, 0)),
                  pl.BlockSpec((B, HID), lambda i: (i, 0)),
                  pl.BlockSpec((B, HID), lambda i: (i, 0)),
                  pl.BlockSpec((B, HID), lambda i: (i, 0)),
                  pl.BlockSpec((B, 8), lambda i: (i, 0)),
                  full((2, HID)), full((HID, HID)), full((HID, HID)),
                  full((HID, HID)), full((1, HID)),
                  full((HID, HID)), full((1, HID)),
                  full((HID, HID)), full((1, HID))],
        out_specs=pl.BlockSpec((B, HID), lambda i: (i, 0)),
        out_shape=jax.ShapeDtypeStruct((N, HID), jnp.float32),
    )(pos_state, h, accA, accS, cntS,
      wp, wh, wu, wm, pup[0]["b"].reshape(1, HID),
      pup[1]["W"], pup[1]["b"].reshape(1, HID),
      pup[2]["W"], pup[2]["b"].reshape(1, HID))


_gather_pos = _make_gather(
    [(0, 2, 0), (1, 2, 1), (1, 3, 0), (1, 3, 1)], 8)
_gather_enc = _make_gather([(0, 1, 0)], HID)
_scatter_plain = _scatter_kernel(False)
_scatter_count = _scatter_kernel(True)


def kernel(h, u, pos_state, pos_action, dis_a2s, dis_s2s, a2s_edge_index,
           s2s_edge_index, params):
    pa_pad = jnp.pad(pos_action, ((0, 0), (0, 6)))
    ps_pad = jnp.pad(pos_state, ((0, 0), (0, 6)))
    zer64 = jnp.zeros((CPAD, HID), jnp.float32)
    zer8 = jnp.zeros((CPAD, 8), jnp.float32)
    ones8 = jnp.zeros((128, 8), jnp.float32).at[:, 0].set(1.0)
    eiA = a2s_edge_index
    eiS = s2s_edge_index

    paA, pdA, psS, pdS = _gather_pos(pa_pad, ps_pad, eiA, eiS)
    enc_u, enc_h = _node_enc_call(u, h, params["u2h_enc_u"],
                                  params["h2h_enc_h"])
    guA, = _gather_enc(enc_u, eiA)
    msgA = _gate_call(paA, pdA, dis_a2s, guA, params["u2h_enc_dis"])
    ghS, = _gather_enc(enc_h, eiS)
    accA, = _scatter_plain(msgA, eiA, zer64)
    msgS = _gate_call(psS, pdS, dis_s2s, ghS, params["h2h_enc_dis"])
    accS, cntS = _scatter_count(msgS, eiS, zer64, zer8, ones8)
    return _final_call(pos_state, h, accA, accS, cntS, params["h_updater"])


# speedup vs baseline: 1.1599x; 1.0087x over previous
import functools

import jax
import jax.numpy as jnp
from jax import lax
from jax.experimental import pallas as pl
from jax.experimental.pallas import tpu as pltpu
from jax.experimental.pallas import tpu_sc as plsc

N = 50000
E = 800000
HID = 64
NC, NS = 2, 16
NW = NC * NS
ROWS = E // 128
SB_FULL = ROWS // 8
TAIL_ROWS = ROWS - SB_FULL * 8
HALF = N // 2
CPAD = 25008
DUMMY = 25000
WB = 1562
GROUP = 8
NGRP = ROWS // GROUP
TAILR = ROWS - NGRP * GROUP

_MESH = dict(core_axis_name="c", subcore_axis_name="s", num_cores=NC,
             num_subcores=NS)


def _make_gather(specs, width, dtype=jnp.float32):
    n_out = len(specs)
    out_type = [jax.ShapeDtypeStruct((E, width), dtype)] * n_out

    @functools.partial(
        pl.kernel, out_type=out_type,
        mesh=plsc.VectorSubcoreMesh(**_MESH),
        compiler_params=pltpu.CompilerParams(use_tc_tiling_on_sc=False),
        scratch_types=[
            pltpu.VMEM((8, 128), jnp.int32),
            pltpu.VMEM((1024, width), dtype),
            pltpu.SemaphoreType.DMA,
        ])
    def k(*refs):
        n_in = max(max(s[0], s[1]) for s in specs) + 1
        ins = refs[:n_in]
        outs = refs[n_in:n_in + n_out]
        idx_v, rows_v, sem = refs[n_in + n_out:]
        cid = lax.axis_index("c")
        sid = lax.axis_index("s")
        wid = sid * NC + cid

        def task(table, ei, row, out):
            def unit(nrows, sb):
                pltpu.sync_copy(ei.at[row, pl.ds(sb * 8, nrows)],
                                idx_v.at[pl.ds(0, nrows)])
                descs = [pltpu.async_copy(
                    table.at[idx_v.at[j]],
                    rows_v.at[pl.ds(j * 128, 128)], sem)
                         for j in range(nrows)]
                for d in descs:
                    d.wait()
                pltpu.sync_copy(rows_v.at[pl.ds(0, nrows * 128)],
                                out.at[pl.ds(sb * 1024, nrows * 128)])

            def body(it, carry):
                sb = it * NW + wid

                @pl.when(sb < SB_FULL)
                def _():
                    unit(8, sb)

                @pl.when(sb == SB_FULL)
                def _():
                    unit(TAIL_ROWS, sb)
                return carry

            lax.fori_loop(0, (SB_FULL + NW) // NW + 1, body, 0)

        for (ti, ei_i, row), out in zip(specs, outs):
            task(ins[ti], ins[ei_i], row, out)

    return k


def _scatter_kernel(with_count):
    out_type = [jax.ShapeDtypeStruct((N, HID), jnp.float32)]
    scratch = [
        pltpu.VMEM((GROUP, 128), jnp.int32),
        pltpu.VMEM((2, 128, HID), jnp.float32),
        pltpu.VMEM_SHARED((CPAD, HID), jnp.float32),
        pltpu.SemaphoreType.DMA,
    ]
    if with_count:
        out_type.append(jax.ShapeDtypeStruct((N, 8), jnp.float32))
        scratch += [pltpu.VMEM((128, 8), jnp.float32),
                    pltpu.VMEM_SHARED((CPAD, 8), jnp.float32)]

    @functools.partial(
        pl.kernel,
        out_type=out_type,
        mesh=plsc.VectorSubcoreMesh(**_MESH),
        compiler_params=pltpu.CompilerParams(use_tc_tiling_on_sc=False),
        scratch_types=scratch)
    def k(msg, ei, zer, *rest):
        if with_count:
            (zer8, ones8, out, out_c, idx_v, msg_v, acc, sem, ones_v,
             acc_c) = rest
        else:
            out, idx_v, msg_v, acc, sem = rest
        cid = lax.axis_index("c")
        sid = lax.axis_index("s")
        base = cid * HALF

        z0 = sid * 1564
        zlast = CPAD - 1564 * (NS - 1)

        @pl.when(sid < NS - 1)
        def _():
            pltpu.sync_copy(zer.at[pl.ds(z0, 1564)], acc.at[pl.ds(z0, 1564)])
            if with_count:
                pltpu.sync_copy(zer8.at[pl.ds(z0, 1564)],
                                acc_c.at[pl.ds(z0, 1564)])

        @pl.when(sid == NS - 1)
        def _():
            pltpu.sync_copy(zer.at[pl.ds(z0, zlast)], acc.at[pl.ds(z0, zlast)])
            if with_count:
                pltpu.sync_copy(zer8.at[pl.ds(z0, zlast)],
                                acc_c.at[pl.ds(z0, zlast)])

        if with_count:
            pltpu.sync_copy(ones8, ones_v)
        plsc.subcore_barrier()

        def unit(nrows, g0):
            pltpu.sync_copy(ei.at[1, pl.ds(g0, nrows)],
                            idx_v.at[pl.ds(0, nrows)])
            for r in range(nrows):
                for c in range(128 // 16):
                    v = idx_v[r, pl.ds(c * 16, 16)]
                    l = v - base
                    ok = (l >= 0) & (l < HALF)
                    idx_v[r, pl.ds(c * 16, 16)] = jnp.where(ok, l, DUMMY)

            def fire(r):
                return pltpu.async_copy(msg.at[pl.ds((g0 + r) * 128, 128)],
                                        msg_v.at[r % 2], sem)

            d = fire(0)
            for r in range(nrows):
                d.wait()
                if r + 1 < nrows:
                    d = fire(r + 1)
                pltpu.sync_copy(msg_v.at[r % 2], acc.at[idx_v.at[r]],
                                add=True)
                if with_count:
                    pltpu.sync_copy(ones_v, acc_c.at[idx_v.at[r]], add=True)

        def body(it, carry):
            g = it * NS + sid

            @pl.when(g < NGRP)
            def _():
                unit(GROUP, g * GROUP)
            return carry

        lax.fori_loop(0, (NGRP + NS - 1) // NS, body, 0)

        @pl.when(sid < TAILR)
        def _():
            unit(1, NGRP * GROUP + sid)

        plsc.subcore_barrier()

        off = sid * WB
        pltpu.sync_copy(acc.at[pl.ds(off, WB)], out.at[pl.ds(base + off, WB)])
        if with_count:
            pltpu.sync_copy(acc_c.at[pl.ds(off, WB)],
                            out_c.at[pl.ds(base + off, WB)])

        @pl.when(sid == NS - 1)
        def _():
            pltpu.sync_copy(acc.at[pl.ds(WB * NS, HALF - WB * NS)],
                            out.at[pl.ds(base + WB * NS, HALF - WB * NS)])
            if with_count:
                pltpu.sync_copy(acc_c.at[pl.ds(WB * NS, HALF - WB * NS)],
                                out_c.at[pl.ds(base + WB * NS, HALF - WB * NS)])

    return k


def _node_enc_call(u, h, pu, ph):
    B = 2000

    def body(u_ref, h_ref, wu1, wu2, wu3, wh1, wh2, wh3, b1, b2, b3,
             eu_ref, eh_ref):
        x = jnp.tanh(jnp.dot(u_ref[...], wu1[...],
                             preferred_element_type=jnp.float32))
        x = jnp.tanh(jnp.dot(x, wu2[...], preferred_element_type=jnp.float32))
        eu_ref[...] = jnp.dot(x, wu3[...], preferred_element_type=jnp.float32)
        y = jnp.tanh(jnp.dot(h_ref[...], wh1[...],
                             preferred_element_type=jnp.float32) + b1[...])
        y = jnp.tanh(jnp.dot(y, wh2[...],
                             preferred_element_type=jnp.float32) + b2[...])
        eh_ref[...] = jnp.dot(y, wh3[...],
                              preferred_element_type=jnp.float32) + b3[...]

    def full(shape):
        return pl.BlockSpec(shape, lambda i: (0,) * len(shape))

    return pl.pallas_call(
        body,
        grid=(N // B,),
        in_specs=[pl.BlockSpec((B, 16), lambda i: (i, 0)),
                  pl.BlockSpec((B, HID), lambda i: (i, 0)),
                  full((16, HID)), full((HID, HID)), full((HID, HID)),
                  full((HID, HID)), full((HID, HID)), full((HID, HID)),
                  full((1, HID)), full((1, HID)), full((1, HID))],
        out_specs=[pl.BlockSpec((B, HID), lambda i: (i, 0))] * 2,
        out_shape=[jax.ShapeDtypeStruct((N, HID), jnp.float32)] * 2,
    )(u, h, pu[0]["W"], pu[1]["W"], pu[2]["W"],
      ph[0]["W"], ph[1]["W"], ph[2]["W"],
      ph[0]["b"].reshape(1, HID), ph[1]["b"].reshape(1, HID),
      ph[2]["b"].reshape(1, HID))


def _gate_call(psrc, pdst, dis, encg, pdis):
    B = 4000

    def body(pa, pd, ds_, eg, w1, b1, w2, b2, w3, b3, out):
        x = (pa[:, 0:1] * w1[0:1, :] + pa[:, 1:2] * w1[1:2, :]
             + pd[:, 0:1] * w1[2:3, :] + pd[:, 1:2] * w1[3:4, :]
             + ds_[...] * w1[4:5, :] + b1[...])
        x = jnp.tanh(x)
        x = jnp.tanh(jnp.dot(x, w2[...],
                             preferred_element_type=jnp.float32) + b2[...])
        g = jax.nn.sigmoid(jnp.dot(x, w3[...],
                                   preferred_element_type=jnp.float32)
                           + b3[...])
        out[...] = g * eg[...]

    def full(shape):
        return pl.BlockSpec(shape, lambda i: (0,) * len(shape))

    return pl.pallas_call(
        body,
        grid=(E // B,),
        in_specs=[pl.BlockSpec((B, 8), lambda i: (i, 0)),
                  pl.BlockSpec((B, 8), lambda i: (i, 0)),
                  pl.BlockSpec((B, 1), lambda i: (i, 0)),
                  pl.BlockSpec((B, HID), lambda i: (i, 0)),
                  full((5, HID)), full((1, HID)),
                  full((HID, HID)), full((1, HID)),
                  full((HID, HID)), full((1, HID))],
        out_specs=pl.BlockSpec((B, HID), lambda i: (i, 0)),
        out_shape=jax.ShapeDtypeStruct((E, HID), jnp.float32),
    )(psrc, pdst, dis, encg,
      pdis[0]["W"], pdis[0]["b"].reshape(1, HID),
      pdis[1]["W"], pdis[1]["b"].reshape(1, HID),
      pdis[2]["W"], pdis[2]["b"].reshape(1, HID))


def _final_call(pos_state, h, accA, accS, cntS, pup):
    B = 2000
    W1 = pup[0]["W"]
    wp, wh = W1[0:2], W1[2:2 + HID]
    wu, wm = W1[2 + HID:2 + 2 * HID], W1[2 + 2 * HID:2 + 3 * HID]

    def body(ps, h_ref, aA, aS, cS, wp_r, wh_r, wu_r, wm_r, b1, w2, b2, w3,
             b3, out):
        cnt = jnp.maximum(cS[:, 0:1], 1.0)
        aSf = aS[...]
        aAf = aA[...]
        mh = aSf / cnt
        x = (ps[:, 0:1] * wp_r[0:1, :] + ps[:, 1:2] * wp_r[1:2, :]
             + jnp.dot(h_ref[...], wh_r[...],
                       preferred_element_type=jnp.float32)
             + jnp.dot(aAf, wu_r[...], preferred_element_type=jnp.float32)
             + jnp.dot(mh, wm_r[...], preferred_element_type=jnp.float32)
             + b1[...])
        x = jnp.tanh(x)
        x = jnp.tanh(jnp.dot(x, w2[...],
                             preferred_element_type=jnp.float32) + b2[...])
        out[...] = jnp.dot(x, w3[...],
                           preferred_element_type=jnp.float32) + b3[...]

    def full(shape):
        return pl.BlockSpec(shape, lambda i: (0,) * len(shape))

    return pl.pallas_call(
        body,
        grid=(N // B,),
        in_specs=[pl.BlockSpec((B, 2), lambda i: (i, 0)),
                  pl.BlockSpec((B, HID), lambda i: (i, 0)),
                  pl.BlockSpec((B, HID), lambda i: (i, 0)),
                  pl.BlockSpec((B, HID), lambda i: (i, 0)),
                  pl.BlockSpec((B, 8), lambda i: (i, 0)),
                  full((2, HID)), full((HID, HID)), full((HID, HID)),
                  full((HID, HID)), full((1, HID)),
                  full((HID, HID)), full((1, HID)),
                  full((HID, HID)), full((1, HID))],
        out_specs=pl.BlockSpec((B, HID), lambda i: (i, 0)),
        out_shape=jax.ShapeDtypeStruct((N, HID), jnp.float32),
    )(pos_state, h, accA, accS, cntS,
      wp, wh, wu, wm, pup[0]["b"].reshape(1, HID),
      pup[1]["W"], pup[1]["b"].reshape(1, HID),
      pup[2]["W"], pup[2]["b"].reshape(1, HID))


_gather_pos = _make_gather(
    [(0, 2, 0), (1, 2, 1), (1, 3, 0), (1, 3, 1)], 8)
_gather_enc = _make_gather([(0, 1, 0)], HID)
_scatter_plain = _scatter_kernel(False)
_scatter_count = _scatter_kernel(True)


def kernel(h, u, pos_state, pos_action, dis_a2s, dis_s2s, a2s_edge_index,
           s2s_edge_index, params):
    pa_pad = jnp.pad(pos_action, ((0, 0), (0, 6)))
    ps_pad = jnp.pad(pos_state, ((0, 0), (0, 6)))
    zer64 = jnp.zeros((CPAD, HID), jnp.float32)
    zer8 = jnp.zeros((CPAD, 8), jnp.float32)
    ones8 = jnp.zeros((128, 8), jnp.float32).at[:, 0].set(1.0)
    eiA = a2s_edge_index.reshape(2, ROWS, 128)
    eiS = s2s_edge_index.reshape(2, ROWS, 128)

    paA, pdA, psS, pdS = _gather_pos(pa_pad, ps_pad, eiA, eiS)
    enc_u, enc_h = _node_enc_call(u, h, params["u2h_enc_u"],
                                  params["h2h_enc_h"])
    guA, = _gather_enc(enc_u, eiA)
    msgA = _gate_call(paA, pdA, dis_a2s, guA, params["u2h_enc_dis"])
    ghS, = _gather_enc(enc_h, eiS)
    accA, = _scatter_plain(msgA, eiA, zer64)
    msgS = _gate_call(psS, pdS, dis_s2s, ghS, params["h2h_enc_dis"])
    accS, cntS = _scatter_count(msgS, eiS, zer64, zer8, ones8)
    return _final_call(pos_state, h, accA, accS, cntS, params["h_updater"])
